# baseline (speedup 1.0000x reference)
import functools

import numpy as np
import jax
import jax.numpy as jnp
from jax import lax
from jax.experimental import pallas as pl
from jax.experimental.pallas import tpu as pltpu

X = 224
Y = 224
B = 4
D = 64
KC = 512
TPC = KC // 128


def _build_tables():
    radius = min(X, Y) / 2
    xc, yc = X / 2, Y / 2
    xs, ys = np.ogrid[:X, :Y]
    mask = np.sqrt((xs - xc) ** 2 + (ys - yc) ** 2) <= radius
    w = mask.sum(axis=1).astype(np.int64)
    y0 = np.array([int(np.argmax(mask[x])) for x in range(X)], dtype=np.int64)
    off = np.concatenate([[0], np.cumsum(w)])[:-1]
    k = int(w.sum())
    nt = -(-k // KC)
    slabmax = 0
    rows = []
    for t in range(nt):
        lo, hi = t * KC, min((t + 1) * KC, k)
        xsel = [x for x in range(X) if off[x] + w[x] > lo and off[x] < hi]
        slabmax = max(slabmax, len(xsel))
        rows.append(xsel)
    tpc = KC // 128
    xlo = np.zeros(nt, np.int32)
    rr_tab = np.zeros((nt, slabmax * tpc), np.int32)
    lo_tab = np.full((nt, slabmax * tpc), 128, np.int32)
    hi_tab = np.zeros((nt, slabmax * tpc), np.int32)
    for t in range(nt):
        xlo[t] = min(rows[t][0], X - slabmax)
        for x in rows[t]:
            s = x - xlo[t]
            cs = max(off[x] - t * KC, 0)
            ce = min(off[x] + w[x] - t * KC, KC)
            d = y0[x] - off[x] + t * KC
            for tc in range(tpc):
                col = s * tpc + tc
                lo_tab[t, col] = max(cs - 128 * tc, 0)
                hi_tab[t, col] = min(ce - 128 * tc, 128)
                rr_tab[t, col] = (-(128 * tc + d)) % Y
    return k, nt, slabmax, xlo, rr_tab, lo_tab, hi_tab


K_ROWS, NT, SLABMAX, _XLO, _RRTAB, _LOTAB, _HITAB = _build_tables()


def _body(xlo_r, rr_r, lo_r, hi_r, tin, out_ref, slabs, sem0, sem1):
    bi = pl.program_id(0)
    ti = pl.program_id(1)
    g = bi * NT + ti
    par = lax.rem(g, 2)

    def _fetch(b, t, buf, sem):
        pltpu.make_async_copy(
            tin.at[b, pl.ds(xlo_r[t], SLABMAX)], slabs.at[buf], sem
        ).start()

    def _wait(buf, sem):
        pltpu.make_async_copy(
            tin.at[0, pl.ds(0, SLABMAX)], slabs.at[buf], sem
        ).wait()

    @pl.when(g == 0)
    def _first():
        _fetch(0, 0, 0, sem0)

    tn = ti + 1
    t_next = jnp.where(tn == NT, 0, tn)
    b_next = jnp.minimum(jnp.where(tn == NT, bi + 1, bi), B - 1)

    @pl.when(jnp.logical_and(par == 0, g + 1 < B * NT))
    def _pf0():
        _fetch(b_next, t_next, 1, sem1)

    @pl.when(jnp.logical_and(par == 1, g + 1 < B * NT))
    def _pf1():
        _fetch(b_next, t_next, 0, sem0)

    @pl.when(par == 0)
    def _w0():
        _wait(0, sem0)

    @pl.when(par == 1)
    def _w1():
        _wait(1, sem1)

    lane = lax.broadcasted_iota(jnp.int32, (D, 128), 1)
    for tc in range(TPC):
        for s in range(SLABMAX):
            col = s * TPC + tc
            lo = lo_r[ti, col]
            hi = hi_r[ti, col]

            @pl.when(hi - lo == 128)
            def _full():
                rolled = pltpu.roll(slabs[par, s], rr_r[ti, col], axis=1)
                out_ref[0, :, pl.ds(128 * tc, 128)] = rolled[:, :128]

            @pl.when(jnp.logical_and(lo < hi, hi - lo < 128))
            def _blend():
                rolled = pltpu.roll(slabs[par, s], rr_r[ti, col], axis=1)
                win = rolled[:, :128]
                m = jnp.logical_and(lane >= lo, lane < hi)
                cur = out_ref[0, :, pl.ds(128 * tc, 128)]
                out_ref[0, :, pl.ds(128 * tc, 128)] = jnp.where(m, win, cur)


@jax.jit
def kernel(tensor):
    tin = jnp.transpose(tensor, (0, 1, 3, 2))
    smem = pl.BlockSpec(memory_space=pltpu.SMEM)
    out3 = pl.pallas_call(
        _body,
        grid=(B, NT),
        in_specs=[
            smem, smem, smem, smem,
            pl.BlockSpec(memory_space=pl.ANY),
        ],
        out_specs=pl.BlockSpec((1, D, KC), lambda b, t: (b, 0, t)),
        out_shape=jax.ShapeDtypeStruct((B, D, K_ROWS), jnp.float32),
        scratch_shapes=[
            pltpu.VMEM((2, SLABMAX, D, Y), jnp.float32),
            pltpu.SemaphoreType.DMA,
            pltpu.SemaphoreType.DMA,
        ],
    )(
        jnp.asarray(_XLO), jnp.asarray(_RRTAB),
        jnp.asarray(_LOTAB), jnp.asarray(_HITAB), tin,
    )
    return jnp.transpose(out3, (0, 2, 1))


# speedup vs baseline: 1.3135x; 1.3135x over previous
import functools

import numpy as np
import jax
import jax.numpy as jnp
from jax import lax
from jax.experimental import pallas as pl
from jax.experimental.pallas import tpu as pltpu

X = 224
Y = 224
B = 4
D = 64
KC = 512
TPC = KC // 128


def _build_tables():
    radius = min(X, Y) / 2
    xc, yc = X / 2, Y / 2
    xs, ys = np.ogrid[:X, :Y]
    mask = np.sqrt((xs - xc) ** 2 + (ys - yc) ** 2) <= radius
    w = mask.sum(axis=1).astype(np.int64)
    y0 = np.array([int(np.argmax(mask[x])) for x in range(X)], dtype=np.int64)
    off = np.concatenate([[0], np.cumsum(w)])[:-1]
    k = int(w.sum())
    nt = -(-k // KC)
    slabmax = 0
    rows = []
    for t in range(nt):
        lo, hi = t * KC, min((t + 1) * KC, k)
        xsel = [x for x in range(X) if off[x] + w[x] > lo and off[x] < hi]
        slabmax = max(slabmax, len(xsel))
        rows.append(xsel)
    tpc = KC // 128
    wt = 0
    tiles = []
    for t in range(nt):
        for tc in range(tpc):
            lo, hi = t * KC + tc * 128, min(t * KC + tc * 128 + 128, k)
            xsel = [x for x in range(X) if lo < hi and off[x] + w[x] > lo
                    and off[x] < hi]
            wt = max(wt, len(xsel))
            tiles.append((t, tc, xsel))
    xlo = np.zeros(nt, np.int32)
    for t in range(nt):
        xlo[t] = min(rows[t][0], X - slabmax)
    s0_tab = np.zeros((nt, tpc), np.int32)
    rr_tab = np.zeros((nt, tpc * wt), np.int32)
    lo_tab = np.full((nt, tpc * wt), 128, np.int32)
    hi_tab = np.zeros((nt, tpc * wt), np.int32)
    for t, tc, xsel in tiles:
        if xsel:
            s0_tab[t, tc] = xsel[0] - xlo[t]
        for j, x in enumerate(xsel):
            col = tc * wt + j
            cs = max(off[x] - t * KC, 0)
            ce = min(off[x] + w[x] - t * KC, KC)
            d = y0[x] - off[x] + t * KC
            lo_tab[t, col] = max(cs - 128 * tc, 0)
            hi_tab[t, col] = min(ce - 128 * tc, 128)
            rr_tab[t, col] = (-(128 * tc + d)) % Y
    return k, nt, slabmax, wt, xlo, s0_tab, rr_tab, lo_tab, hi_tab


(K_ROWS, NT, SLABMAX, WT, _XLO, _S0TAB, _RRTAB, _LOTAB,
 _HITAB) = _build_tables()


def _body(xlo_r, s0_r, rr_r, lo_r, hi_r, tin, out_ref, slabs, sem0, sem1):
    bi = pl.program_id(0)
    ti = pl.program_id(1)
    g = bi * NT + ti
    par = lax.rem(g, 2)

    def _fetch(b, t, buf, sem):
        pltpu.make_async_copy(
            tin.at[b, pl.ds(xlo_r[t], SLABMAX)], slabs.at[buf], sem
        ).start()

    def _wait(buf, sem):
        pltpu.make_async_copy(
            tin.at[0, pl.ds(0, SLABMAX)], slabs.at[buf], sem
        ).wait()

    @pl.when(g == 0)
    def _first():
        _fetch(0, 0, 0, sem0)

    tn = ti + 1
    t_next = jnp.where(tn == NT, 0, tn)
    b_next = jnp.minimum(jnp.where(tn == NT, bi + 1, bi), B - 1)

    @pl.when(jnp.logical_and(par == 0, g + 1 < B * NT))
    def _pf0():
        _fetch(b_next, t_next, 1, sem1)

    @pl.when(jnp.logical_and(par == 1, g + 1 < B * NT))
    def _pf1():
        _fetch(b_next, t_next, 0, sem0)

    @pl.when(par == 0)
    def _w0():
        _wait(0, sem0)

    @pl.when(par == 1)
    def _w1():
        _wait(1, sem1)

    lane = lax.broadcasted_iota(jnp.int32, (D, 128), 1)
    for tc in range(TPC):
        s0 = s0_r[ti, tc]
        for j in range(WT):
            col = tc * WT + j
            lo = lo_r[ti, col]
            hi = hi_r[ti, col]

            @pl.when(lo < hi)
            def _blend():
                rolled = pltpu.roll(slabs[par, s0 + j], rr_r[ti, col], axis=1)
                win = rolled[:, :128]
                m = jnp.logical_and(lane >= lo, lane < hi)
                cur = out_ref[0, :, pl.ds(128 * tc, 128)]
                out_ref[0, :, pl.ds(128 * tc, 128)] = jnp.where(m, win, cur)


@jax.jit
def kernel(tensor):
    tin = jnp.transpose(tensor, (0, 1, 3, 2))
    smem = pl.BlockSpec(memory_space=pltpu.SMEM)
    out3 = pl.pallas_call(
        _body,
        grid=(B, NT),
        in_specs=[
            smem, smem, smem, smem, smem,
            pl.BlockSpec(memory_space=pl.ANY),
        ],
        out_specs=pl.BlockSpec((1, D, KC), lambda b, t: (b, 0, t)),
        out_shape=jax.ShapeDtypeStruct((B, D, K_ROWS), jnp.float32),
        scratch_shapes=[
            pltpu.VMEM((2, SLABMAX, D, Y), jnp.float32),
            pltpu.SemaphoreType.DMA,
            pltpu.SemaphoreType.DMA,
        ],
    )(
        jnp.asarray(_XLO), jnp.asarray(_S0TAB), jnp.asarray(_RRTAB),
        jnp.asarray(_LOTAB), jnp.asarray(_HITAB), tin,
    )
    return jnp.transpose(out3, (0, 2, 1))


# speedup vs baseline: 1.3638x; 1.0383x over previous
import functools

import numpy as np
import jax
import jax.numpy as jnp
from jax import lax
from jax.experimental import pallas as pl
from jax.experimental.pallas import tpu as pltpu

X = 224
Y = 224
B = 4
D = 64
KC = 1024
TPC = KC // 128


def _build_tables():
    radius = min(X, Y) / 2
    xc, yc = X / 2, Y / 2
    xs, ys = np.ogrid[:X, :Y]
    mask = np.sqrt((xs - xc) ** 2 + (ys - yc) ** 2) <= radius
    w = mask.sum(axis=1).astype(np.int64)
    y0 = np.array([int(np.argmax(mask[x])) for x in range(X)], dtype=np.int64)
    off = np.concatenate([[0], np.cumsum(w)])[:-1]
    k = int(w.sum())
    nt = -(-k // KC)
    slabmax = 0
    rows = []
    for t in range(nt):
        lo, hi = t * KC, min((t + 1) * KC, k)
        xsel = [x for x in range(X) if off[x] + w[x] > lo and off[x] < hi]
        slabmax = max(slabmax, len(xsel))
        rows.append(xsel)
    tpc = KC // 128
    wt = 0
    tiles = []
    for t in range(nt):
        for tc in range(tpc):
            lo, hi = t * KC + tc * 128, min(t * KC + tc * 128 + 128, k)
            xsel = [x for x in range(X) if lo < hi and off[x] + w[x] > lo
                    and off[x] < hi]
            wt = max(wt, len(xsel))
            tiles.append((t, tc, xsel))
    xlo = np.zeros(nt, np.int32)
    for t in range(nt):
        xlo[t] = min(rows[t][0], X - slabmax)
    s0_tab = np.zeros((nt, tpc), np.int32)
    rr_tab = np.zeros((nt, tpc * wt), np.int32)
    lo_tab = np.full((nt, tpc * wt), 128, np.int32)
    hi_tab = np.zeros((nt, tpc * wt), np.int32)
    for t, tc, xsel in tiles:
        if xsel:
            s0_tab[t, tc] = xsel[0] - xlo[t]
        for j, x in enumerate(xsel):
            col = tc * wt + j
            cs = max(off[x] - t * KC, 0)
            ce = min(off[x] + w[x] - t * KC, KC)
            d = y0[x] - off[x] + t * KC
            lo_tab[t, col] = max(cs - 128 * tc, 0)
            hi_tab[t, col] = min(ce - 128 * tc, 128)
            rr_tab[t, col] = (-(128 * tc + d)) % Y
    return k, nt, slabmax, wt, xlo, s0_tab, rr_tab, lo_tab, hi_tab


(K_ROWS, NT, SLABMAX, WT, _XLO, _S0TAB, _RRTAB, _LOTAB,
 _HITAB) = _build_tables()


def _body(xlo_r, s0_r, rr_r, lo_r, hi_r, tin, out_ref, slabs, sem0, sem1):
    bi = pl.program_id(0)
    ti = pl.program_id(1)
    g = bi * NT + ti
    par = lax.rem(g, 2)

    def _fetch(b, t, buf, sem):
        pltpu.make_async_copy(
            tin.at[b, pl.ds(xlo_r[t], SLABMAX)], slabs.at[buf], sem
        ).start()

    def _wait(buf, sem):
        pltpu.make_async_copy(
            tin.at[0, pl.ds(0, SLABMAX)], slabs.at[buf], sem
        ).wait()

    @pl.when(g == 0)
    def _first():
        _fetch(0, 0, 0, sem0)

    tn = ti + 1
    t_next = jnp.where(tn == NT, 0, tn)
    b_next = jnp.minimum(jnp.where(tn == NT, bi + 1, bi), B - 1)

    @pl.when(jnp.logical_and(par == 0, g + 1 < B * NT))
    def _pf0():
        _fetch(b_next, t_next, 1, sem1)

    @pl.when(jnp.logical_and(par == 1, g + 1 < B * NT))
    def _pf1():
        _fetch(b_next, t_next, 0, sem0)

    @pl.when(par == 0)
    def _w0():
        _wait(0, sem0)

    @pl.when(par == 1)
    def _w1():
        _wait(1, sem1)

    lane = lax.broadcasted_iota(jnp.int32, (D, 128), 1)
    for tc in range(TPC):
        s0 = s0_r[ti, tc]
        for j in range(WT):
            col = tc * WT + j
            lo = lo_r[ti, col]
            hi = hi_r[ti, col]

            @pl.when(lo < hi)
            def _blend():
                rolled = pltpu.roll(slabs[par, s0 + j], rr_r[ti, col], axis=1)
                win = rolled[:, :128]
                m = jnp.logical_and(lane >= lo, lane < hi)
                cur = out_ref[0, :, pl.ds(128 * tc, 128)]
                out_ref[0, :, pl.ds(128 * tc, 128)] = jnp.where(m, win, cur)


@jax.jit
def kernel(tensor):
    tin = jnp.transpose(tensor, (0, 1, 3, 2))
    smem = pl.BlockSpec(memory_space=pltpu.SMEM)
    out3 = pl.pallas_call(
        _body,
        grid=(B, NT),
        in_specs=[
            smem, smem, smem, smem, smem,
            pl.BlockSpec(memory_space=pl.ANY),
        ],
        out_specs=pl.BlockSpec((1, D, KC), lambda b, t: (b, 0, t)),
        out_shape=jax.ShapeDtypeStruct((B, D, K_ROWS), jnp.float32),
        scratch_shapes=[
            pltpu.VMEM((2, SLABMAX, D, Y), jnp.float32),
            pltpu.SemaphoreType.DMA,
            pltpu.SemaphoreType.DMA,
        ],
    )(
        jnp.asarray(_XLO), jnp.asarray(_S0TAB), jnp.asarray(_RRTAB),
        jnp.asarray(_LOTAB), jnp.asarray(_HITAB), tin,
    )
    return jnp.transpose(out3, (0, 2, 1))
